# baseline (speedup 1.0000x reference)
import functools

import jax
import jax.numpy as jnp
from jax import lax
from jax.experimental import pallas as pl
from jax.experimental.pallas import tpu as pltpu
from jax.experimental.pallas import tpu_sc as plsc

DEPTH = 1000
D_CH = 200
NC0 = 128


def kernel(inputs):
    B1, B2 = inputs.shape
    NC, NS = 2, 16
    assert B1 % NS == 0 and DEPTH % D_CH == 0
    ipw = B1 // NS
    ND = DEPTH // D_CH
    total_chunks = B2 * ND
    assert NC0 % 2 == 0 and (total_chunks - NC0) % 2 == 0

    mesh = plsc.VectorSubcoreMesh(core_axis_name="c", subcore_axis_name="s")

    @functools.partial(
        pl.kernel,
        mesh=mesh,
        compiler_params=pltpu.CompilerParams(
            use_tc_tiling_on_sc=True, needs_layout_passes=False
        ),
        out_type=jax.ShapeDtypeStruct((B2, DEPTH, B1), jnp.float32),
        scratch_types=[
            pltpu.VMEM((B2, ipw), jnp.int32),
            pltpu.VMEM((D_CH, ipw), jnp.float32),
            pltpu.VMEM((D_CH, ipw), jnp.float32),
            pltpu.SemaphoreType.DMA,
            pltpu.SemaphoreType.DMA,
        ],
    )
    def onehot_sc(idx_hbm, out_hbm, idx_v, buf_a, buf_b, sem_a, sem_b):
        c = lax.axis_index("c")
        s = lax.axis_index("s")
        i0 = s * ipw
        t_base = jnp.where(c == 0, 0, NC0)
        nch = jnp.where(c == 0, NC0, total_chunks - NC0)

        pltpu.sync_copy(idx_hbm.at[:, pl.ds(i0, ipw)], idx_v)

        lanes = lax.iota(jnp.int32, 16)
        zeros_v = jnp.zeros((16,), jnp.float32)
        ones_v = jnp.ones((16,), jnp.float32)

        def scatter_chunk(buf, t, val_vec):
            j = t // ND
            d0 = (t % ND) * D_CH
            for g in range(ipw // 16):
                kv = idx_v[j, pl.ds(g * 16, 16)]
                m = (kv >= d0) & (kv < d0 + D_CH)
                row = jnp.clip(kv - d0, 0, D_CH - 1)
                plsc.store_scatter(buf, [row, lanes + g * 16], val_vec, mask=m)

        def start_dma(buf, t, sem):
            j = t // ND
            d0 = pl.multiple_of((t % ND) * D_CH, 8)
            pltpu.async_copy(
                buf, out_hbm.at[j, pl.ds(d0, D_CH), pl.ds(i0, ipw)], sem
            )

        def wait_dma(buf, sem):
            pltpu.make_async_copy(
                buf, out_hbm.at[0, pl.ds(0, D_CH), pl.ds(i0, ipw)], sem
            ).wait()

        def zrow(d, _):
            for g in range(ipw // 16):
                buf_a[d, pl.ds(g * 16, 16)] = zeros_v
                buf_b[d, pl.ds(g * 16, 16)] = zeros_v
            return 0

        lax.fori_loop(0, D_CH, zrow, 0)

        scatter_chunk(buf_a, t_base, ones_v)
        start_dma(buf_a, t_base, sem_a)
        scatter_chunk(buf_b, t_base + 1, ones_v)
        start_dma(buf_b, t_base + 1, sem_b)

        def body(rr, _):
            for off, buf, sem in ((2, buf_a, sem_a), (3, buf_b, sem_b)):
                t = t_base + 2 * rr + off
                wait_dma(buf, sem)
                scatter_chunk(buf, t - 2, zeros_v)
                scatter_chunk(buf, t, ones_v)
                start_dma(buf, t, sem)
            return 0

        lax.fori_loop(0, (nch - 2) // 2, body, 0)

        wait_dma(buf_a, sem_a)
        wait_dma(buf_b, sem_b)

    out_t = onehot_sc(inputs.T)
    return jnp.transpose(out_t, (2, 0, 1))


# speedup vs baseline: 1.0499x; 1.0499x over previous
import functools

import jax
import jax.numpy as jnp
from jax import lax
from jax.experimental import pallas as pl
from jax.experimental.pallas import tpu as pltpu
from jax.experimental.pallas import tpu_sc as plsc

DEPTH = 1000
D_CH = 200


def kernel(inputs):
    B1, B2 = inputs.shape
    NC, NS = 2, 16
    NW = NC * NS
    assert B1 % NW == 0 and DEPTH % D_CH == 0
    ipw = B1 // NW
    ND = DEPTH // D_CH
    n_chunks = B2 * ND

    mesh = plsc.VectorSubcoreMesh(core_axis_name="c", subcore_axis_name="s")

    @functools.partial(
        pl.kernel,
        mesh=mesh,
        compiler_params=pltpu.CompilerParams(
            use_tc_tiling_on_sc=True, needs_layout_passes=False
        ),
        out_type=jax.ShapeDtypeStruct((B2, DEPTH, B1), jnp.float32),
        scratch_types=[
            pltpu.VMEM((B2, ipw), jnp.int32),
            pltpu.VMEM((D_CH, ipw), jnp.float32),
            pltpu.VMEM((D_CH, ipw), jnp.float32),
            pltpu.SemaphoreType.DMA,
            pltpu.SemaphoreType.DMA,
        ],
    )
    def onehot_sc(idx_hbm, out_hbm, idx_v, buf_a, buf_b, sem_a, sem_b):
        c = lax.axis_index("c")
        s = lax.axis_index("s")
        wid = s * NC + c
        i0 = wid * ipw

        pltpu.sync_copy(idx_hbm.at[:, pl.ds(i0, ipw)], idx_v)

        lanes = lax.iota(jnp.int32, 16)
        zeros_v = jnp.zeros((16,), jnp.float32)
        ones_v = jnp.ones((16,), jnp.float32)

        def scatter_chunk(buf, t, val_vec):
            j = t // ND
            d0 = (t % ND) * D_CH
            for g in range(ipw // 16):
                kv = idx_v[j, pl.ds(g * 16, 16)]
                m = (kv >= d0) & (kv < d0 + D_CH)
                row = jnp.clip(kv - d0, 0, D_CH - 1)
                plsc.store_scatter(buf, [row, lanes + g * 16], val_vec, mask=m)

        def start_dma(buf, t, sem):
            j = t // ND
            d0 = pl.multiple_of((t % ND) * D_CH, 8)
            pltpu.async_copy(
                buf, out_hbm.at[j, pl.ds(d0, D_CH), pl.ds(i0, ipw)], sem
            )

        def wait_dma(buf, sem):
            pltpu.make_async_copy(
                buf, out_hbm.at[0, pl.ds(0, D_CH), pl.ds(i0, ipw)], sem
            ).wait()

        def zrow(d, _):
            for g in range(ipw // 16):
                buf_a[d, pl.ds(g * 16, 16)] = zeros_v
                buf_b[d, pl.ds(g * 16, 16)] = zeros_v
            return 0

        lax.fori_loop(0, D_CH, zrow, 0)

        scatter_chunk(buf_a, 0, ones_v)
        start_dma(buf_a, 0, sem_a)
        scatter_chunk(buf_b, 1, ones_v)
        start_dma(buf_b, 1, sem_b)

        def body(rr, _):
            for off, buf, sem in ((2, buf_a, sem_a), (3, buf_b, sem_b)):
                t = 2 * rr + off
                wait_dma(buf, sem)
                scatter_chunk(buf, t - 2, zeros_v)
                scatter_chunk(buf, t, ones_v)
                start_dma(buf, t, sem)
            return 0

        lax.fori_loop(0, (n_chunks - 2) // 2, body, 0)

        wait_dma(buf_a, sem_a)
        wait_dma(buf_b, sem_b)

    out_t = onehot_sc(inputs.T)
    return jnp.transpose(out_t, (2, 0, 1))
